# baseline (speedup 1.0000x reference)
import jax
import jax.numpy as jnp
from jax.experimental import pallas as pl

N_NODES = 50000
LATENT_DIM = 64


def _mean4_body(a_ref, b_ref, c_ref, d_ref, o_ref):
    o_ref[...] = 0.25 * (a_ref[...] + b_ref[...] + c_ref[...] + d_ref[...])


def _mean4(a, b, c, d):
    blk = 1000
    grid = (N_NODES // blk,)
    spec = pl.BlockSpec((blk, LATENT_DIM), lambda i: (i, 0))
    return pl.pallas_call(
        _mean4_body,
        grid=grid,
        in_specs=[spec, spec, spec, spec],
        out_specs=spec,
        out_shape=jax.ShapeDtypeStruct((N_NODES, LATENT_DIM), jnp.float32),
    )(a, b, c, d)


def kernel(user_emb, item_emb, edge_weight, edge_index, stages):
    emb0 = jnp.concatenate([user_emb, item_emb], axis=0)
    perm = jnp.argsort(edge_index[0])
    row = jnp.take(edge_index[0], perm)
    col = jnp.take(edge_index[1], perm)
    edge_weight = jnp.take(edge_weight, perm)
    emb = emb0
    embs = [emb0]
    for _ in range(3):
        msgs = jnp.take(emb, col, axis=0) * edge_weight[:, None]
        emb = jax.ops.segment_sum(msgs, row, num_segments=N_NODES)
        embs.append(emb)
    out = _mean4(*embs)
    return out, emb0


# speedup vs baseline: 4.2127x; 4.2127x over previous
import functools

import jax
import jax.numpy as jnp
from jax import lax
from jax.experimental import pallas as pl
from jax.experimental.pallas import tpu as pltpu
from jax.experimental.pallas import tpu_sc as plsc

N_NODES = 50000
LATENT_DIM = 64
N_EDGES = 800000

NC, NS = 2, 16
NW = NC * NS
R_NODES = 1568
N_PAD = NW * R_NODES
SB = 1024
KB = 128
NBLK = SB // KB
E_PAD = N_EDGES + 2 * SB

_mesh = plsc.VectorSubcoreMesh(
    core_axis_name="c", subcore_axis_name="s", num_cores=NC, num_subcores=NS
)


def _stage_body(emb_in, col_hbm, wt_hbm, row_hbm, off_hbm, emb_out,
                acc, colv, wv, rv, offv, rows0, rows1, sem0, sem1):
    wid = lax.axis_index("s") * NC + lax.axis_index("c")
    base = wid * R_NODES

    def _zero(r, _):
        for cc in range(4):
            acc[r, pl.ds(cc * 16, 16)] = jnp.zeros((16,), jnp.float32)
        return 0
    lax.fori_loop(0, R_NODES + 1, _zero, 0)

    pltpu.sync_copy(off_hbm.at[pl.ds(wid * 8, 16)], offv)
    offvec = offv[pl.ds(0, 16)]
    start = offvec[0]
    end = offvec[1]
    start_al = (start // 8) * 8
    nsb = (end - start_al + SB - 1) // SB

    rbufs = (rows0, rows1)
    sems = (sem0, sem1)

    def _edge_group(ebase, elocal, rbuf):
        dstv = rv[pl.ds(ebase, 16)] - base
        dstv = jnp.where((dstv >= 0) & (dstv < R_NODES), dstv, R_NODES)
        wvv = wv[pl.ds(ebase, 16)]
        for u in range(16):
            dl = dstv[u]
            wb = jnp.full((16,), wvv[u], jnp.float32)
            el = elocal + u
            for cc in range(4):
                x = rbuf[el, pl.ds(cc * 16, 16)]
                plsc.addupdate(acc.at[dl, pl.ds(cc * 16, 16)], x * wb)

    def _sb_body(sb, _):
        e0 = start_al + sb * SB
        pltpu.sync_copy(col_hbm.at[pl.ds(e0, SB)], colv)
        pltpu.sync_copy(wt_hbm.at[pl.ds(e0, SB)], wv)
        pltpu.sync_copy(row_hbm.at[pl.ds(e0, SB)], rv)
        pltpu.make_async_copy(
            emb_in.at[colv.at[pl.ds(0, KB)]], rbufs[0], sems[0]).start()
        for b in range(NBLK):
            if b + 1 < NBLK:
                pltpu.make_async_copy(
                    emb_in.at[colv.at[pl.ds((b + 1) * KB, KB)]],
                    rbufs[(b + 1) % 2], sems[(b + 1) % 2]).start()
            pltpu.make_async_copy(
                emb_in.at[colv.at[pl.ds(b * KB, KB)]],
                rbufs[b % 2], sems[b % 2]).wait()

            def _blk(eq, _, _b=b):
                _edge_group(_b * KB + eq * 16, eq * 16, rbufs[_b % 2])
                return 0
            lax.fori_loop(0, KB // 16, _blk, 0)
        return 0

    lax.fori_loop(0, nsb, _sb_body, 0)

    pltpu.sync_copy(acc.at[pl.ds(0, R_NODES)], emb_out.at[pl.ds(base, R_NODES)])


_stage = pl.kernel(
    _stage_body,
    out_type=jax.ShapeDtypeStruct((N_PAD, LATENT_DIM), jnp.float32),
    mesh=_mesh,
    scratch_types=[
        pltpu.VMEM((R_NODES + 1, LATENT_DIM), jnp.float32),
        pltpu.VMEM((SB,), jnp.int32),
        pltpu.VMEM((SB,), jnp.float32),
        pltpu.VMEM((SB,), jnp.int32),
        pltpu.VMEM((16,), jnp.int32),
        pltpu.VMEM((KB, LATENT_DIM), jnp.float32),
        pltpu.VMEM((KB, LATENT_DIM), jnp.float32),
        pltpu.SemaphoreType.DMA,
        pltpu.SemaphoreType.DMA,
    ],
    compiler_params=pltpu.CompilerParams(use_tc_tiling_on_sc=False),
)


def _mean4_body(a_ref, b_ref, c_ref, d_ref, o_ref):
    o_ref[...] = 0.25 * (a_ref[...] + b_ref[...] + c_ref[...] + d_ref[...])


def _mean4(a, b, c, d):
    blk = 400
    spec = pl.BlockSpec((blk, LATENT_DIM), lambda i: (i, 0))
    return pl.pallas_call(
        _mean4_body,
        grid=(N_NODES // blk,),
        in_specs=[spec, spec, spec, spec],
        out_specs=spec,
        out_shape=jax.ShapeDtypeStruct((N_NODES, LATENT_DIM), jnp.float32),
    )(a, b, c, d)


def kernel(user_emb, item_emb, edge_weight, edge_index, stages):
    emb0 = jnp.concatenate([user_emb, item_emb], axis=0)
    row = edge_index[0].astype(jnp.int32)
    col = edge_index[1].astype(jnp.int32)

    perm = jnp.argsort(row)
    row_s = jnp.take(row, perm)
    col_s = jnp.take(col, perm)
    wt_s = jnp.take(edge_weight, perm)
    bounds = (jnp.arange(NW + 1, dtype=jnp.int32) * R_NODES)
    off = jnp.searchsorted(row_s, bounds, side="left").astype(jnp.int32)
    off_pairs = jnp.zeros((NW * 8 + 8,), jnp.int32)
    off_pairs = off_pairs.at[0 : NW * 8 : 8].set(off[:NW])
    off_pairs = off_pairs.at[1 : NW * 8 : 8].set(off[1 : NW + 1])

    pad_e = E_PAD - N_EDGES
    row_s = jnp.pad(row_s, (0, pad_e), constant_values=jnp.int32(2**20))
    col_s = jnp.pad(col_s, (0, pad_e))
    wt_s = jnp.pad(wt_s, (0, pad_e))

    emb = jnp.pad(emb0, ((0, N_PAD - N_NODES), (0, 0)))
    embs = [emb]
    for _ in range(3):
        emb = _stage(emb, col_s, wt_s, row_s, off_pairs)
        embs.append(emb)
    out = _mean4(*embs)
    return out, emb0


# speedup vs baseline: 5.2517x; 1.2467x over previous
import jax
import jax.numpy as jnp
from jax import lax
from jax.experimental import pallas as pl
from jax.experimental.pallas import tpu as pltpu
from jax.experimental.pallas import tpu_sc as plsc

N_NODES = 50000
LATENT_DIM = 64
N_EDGES = 800000

NC, NS = 2, 16
R_NODES = 1568
HALF = NS * R_NODES
N_PAD = NC * HALF
SB = 512
KB = 64
NBLK = SB // KB
NSB = 98
EC = NSB * SB
E_PAD = NS * EC
ZR = 49

_mesh = plsc.VectorSubcoreMesh(
    core_axis_name="c", subcore_axis_name="s", num_cores=NC, num_subcores=NS
)


def _stage_body(emb_in, col_hbm, wt_hbm, row_hbm, emb_out,
                acc, colv, wv, rv, idxb, zbuf, rows0, rows1, sc0, sc1,
                sem0, sem1, ssem0, ssem1):
    cid = lax.axis_index("c")
    sid = lax.axis_index("s")
    base = cid * HALF

    def _zrow(r, _):
        for cc in range(4):
            zbuf[r, pl.ds(cc * 16, 16)] = jnp.zeros((16,), jnp.float32)
        return 0
    lax.fori_loop(0, ZR, _zrow, 0)
    for z in range(R_NODES // ZR):
        pltpu.sync_copy(zbuf, acc.at[pl.ds(sid * R_NODES + z * ZR, ZR)])

    @pl.when(sid == 0)
    def _():
        pltpu.sync_copy(zbuf.at[pl.ds(0, 8)], acc.at[pl.ds(HALF, 8)])

    plsc.subcore_barrier()

    rbufs = (rows0, rows1)
    sbufs = (sc0, sc1)
    sems = (sem0, sem1)
    ssems = (ssem0, ssem1)
    chunk0 = sid * EC

    def _sb_body(sb, _):
        e0 = chunk0 + sb * SB
        pltpu.sync_copy(col_hbm.at[pl.ds(e0, SB)], colv)
        pltpu.sync_copy(wt_hbm.at[pl.ds(e0, SB)], wv)
        pltpu.sync_copy(row_hbm.at[pl.ds(e0, SB)], rv)

        @plsc.parallel_loop(0, SB // 16)
        def _idx(g):
            dstv = rv[pl.ds(g * 16, 16)] - base
            dstv = jnp.where((dstv >= 0) & (dstv < HALF), dstv, HALF)
            idxb[g // (KB // 16), pl.ds((g % (KB // 16)) * 16, 16)] = dstv

        pltpu.make_async_copy(
            emb_in.at[colv.at[pl.ds(0, KB)]], rbufs[0], sems[0]).start()
        for b in range(NBLK):
            if b + 1 < NBLK:
                pltpu.make_async_copy(
                    emb_in.at[colv.at[pl.ds((b + 1) * KB, KB)]],
                    rbufs[(b + 1) % 2], sems[(b + 1) % 2]).start()
            pltpu.make_async_copy(
                emb_in.at[colv.at[pl.ds(b * KB, KB)]],
                rbufs[b % 2], sems[b % 2]).wait()
            if b >= 2:
                pltpu.make_async_copy(
                    sbufs[b % 2], acc.at[idxb.at[b - 2]],
                    ssems[b % 2]).wait()

            rbuf = rbufs[b % 2]
            sbuf = sbufs[b % 2]

            @plsc.parallel_loop(0, KB // 16, unroll=2)
            def _mul(g, _b=b):
                wvv = wv[pl.ds(_b * KB + g * 16, 16)]
                for u in range(16):
                    wb = jnp.full((16,), wvv[u], jnp.float32)
                    el = g * 16 + u
                    for cc in range(4):
                        sl = pl.ds(cc * 16, 16)
                        sbuf[el, sl] = rbuf[el, sl] * wb

            pltpu.async_copy(sbuf, acc.at[idxb.at[b]],
                             ssems[b % 2], add=True)
        pltpu.make_async_copy(
            sbufs[(NBLK - 2) % 2], acc.at[idxb.at[NBLK - 2]],
            ssems[(NBLK - 2) % 2]).wait()
        pltpu.make_async_copy(
            sbufs[(NBLK - 1) % 2], acc.at[idxb.at[NBLK - 1]],
            ssems[(NBLK - 1) % 2]).wait()
        return 0

    lax.fori_loop(0, NSB, _sb_body, 0)
    plsc.subcore_barrier()

    pltpu.sync_copy(acc.at[pl.ds(sid * R_NODES, R_NODES)],
                    emb_out.at[pl.ds(base + sid * R_NODES, R_NODES)])


_stage = pl.kernel(
    _stage_body,
    out_type=jax.ShapeDtypeStruct((N_PAD, LATENT_DIM), jnp.float32),
    mesh=_mesh,
    scratch_types=[
        pltpu.VMEM_SHARED((HALF + 8, LATENT_DIM), jnp.float32),
        pltpu.VMEM((SB,), jnp.int32),
        pltpu.VMEM((SB,), jnp.float32),
        pltpu.VMEM((SB,), jnp.int32),
        pltpu.VMEM((NBLK, KB), jnp.int32),
        pltpu.VMEM((ZR, LATENT_DIM), jnp.float32),
        pltpu.VMEM((KB, LATENT_DIM), jnp.float32),
        pltpu.VMEM((KB, LATENT_DIM), jnp.float32),
        pltpu.VMEM((KB, LATENT_DIM), jnp.float32),
        pltpu.VMEM((KB, LATENT_DIM), jnp.float32),
        pltpu.SemaphoreType.DMA,
        pltpu.SemaphoreType.DMA,
        pltpu.SemaphoreType.DMA,
        pltpu.SemaphoreType.DMA,
    ],
    compiler_params=pltpu.CompilerParams(use_tc_tiling_on_sc=False),
)


def _mean4_body(a_ref, b_ref, c_ref, d_ref, o_ref):
    o_ref[...] = 0.25 * (a_ref[...] + b_ref[...] + c_ref[...] + d_ref[...])


def _mean4(a, b, c, d):
    blk = 400
    spec = pl.BlockSpec((blk, LATENT_DIM), lambda i: (i, 0))
    return pl.pallas_call(
        _mean4_body,
        grid=(N_NODES // blk,),
        in_specs=[spec, spec, spec, spec],
        out_specs=spec,
        out_shape=jax.ShapeDtypeStruct((N_NODES, LATENT_DIM), jnp.float32),
    )(a, b, c, d)


def kernel(user_emb, item_emb, edge_weight, edge_index, stages):
    emb0 = jnp.concatenate([user_emb, item_emb], axis=0)
    row = edge_index[0].astype(jnp.int32)
    col = edge_index[1].astype(jnp.int32)

    pad_e = E_PAD - N_EDGES
    row_p = jnp.pad(row, (0, pad_e), constant_values=jnp.int32(2**20))
    col_p = jnp.pad(col, (0, pad_e))
    wt_p = jnp.pad(edge_weight, (0, pad_e))

    emb = jnp.pad(emb0, ((0, N_PAD - N_NODES), (0, 0)))
    embs = [emb]
    for _ in range(3):
        emb = _stage(emb, col_p, wt_p, row_p)
        embs.append(emb)
    out = _mean4(*embs)
    return out, emb0
